# Initial kernel scaffold; baseline (speedup 1.0000x reference)
import functools
import jax
import jax.numpy as jnp
from jax import lax
from jax.experimental import pallas as pl
from jax.experimental.pallas import tpu as pltpu
from jax.experimental.pallas import tpu_sc as plsc

_B, _H, _W, _C = 4, 384, 384, 96
_HW = _H * _W
_N = _B * _HW
_NC, _NS = 2, 16
_NTILES = _NC * _NS
_PPT = _N // _NTILES
_CHUNK = 128
_STEPS = _PPT // _CHUNK
_GROUPS = _C // 16


def _sc_body(table, samp, out, samp_v, idx0, idx1, idx2, idx3,
             fx_v, fy_v, r0, r1, r2, r3, o_v, sem):
    wid = lax.axis_index("s") * _NC + lax.axis_index("c")
    row_base = (wid // 8) * _HW
    p0 = wid * _PPT
    iota = lax.iota(jnp.int32, 16)

    def step(t, carry):
        prow = p0 + t * _CHUNK
        pltpu.sync_copy(samp.at[pl.ds(prow * 2, _CHUNK * 2)], samp_v)
        for j in range(_CHUNK // 16):
            sx = plsc.load_gather(samp_v, [iota * 2 + (32 * j)])
            sy = plsc.load_gather(samp_v, [iota * 2 + (32 * j + 1)])
            xs = (sx + 1.0) * (0.5 * (_W - 1))
            ys = (sy + 1.0) * (0.5 * (_H - 1))
            xi = jnp.minimum(xs.astype(jnp.int32), _W - 2)
            yi = jnp.minimum(ys.astype(jnp.int32), _H - 2)
            fx = xs - xi.astype(jnp.float32)
            fy = ys - yi.astype(jnp.float32)
            base = row_base + yi * _W + xi
            sl = pl.ds(16 * j, 16)
            idx0[sl] = base
            idx1[sl] = base + 1
            idx2[sl] = base + _W
            idx3[sl] = base + (_W + 1)
            fx_v[sl] = fx
            fy_v[sl] = fy
        cp0 = pltpu.async_copy(table.at[idx0], r0, sem)
        cp1 = pltpu.async_copy(table.at[idx1], r1, sem)
        cp2 = pltpu.async_copy(table.at[idx2], r2, sem)
        cp3 = pltpu.async_copy(table.at[idx3], r3, sem)
        cp0.wait()
        cp1.wait()
        cp2.wait()
        cp3.wait()

        def pix(i, c):
            fxv = plsc.load_gather(fx_v, [jnp.full((16,), i, jnp.int32)])
            fyv = plsc.load_gather(fy_v, [jnp.full((16,), i, jnp.int32)])
            for g in range(_GROUPS):
                gs = pl.ds(g * 16, 16)
                a0 = r0[i, gs]
                a1 = r1[i, gs]
                a2 = r2[i, gs]
                a3 = r3[i, gs]
                top = a0 + fxv * (a1 - a0)
                bot = a2 + fxv * (a3 - a2)
                o_v[i, gs] = top + fyv * (bot - top)
            return c

        lax.fori_loop(0, _CHUNK, pix, 0, unroll=2)
        pltpu.sync_copy(o_v, out.at[pl.ds(prow, _CHUNK)])
        return carry

    lax.fori_loop(0, _STEPS, step, 0)


@jax.jit
def kernel(images, sampling):
    table = images.reshape(_N, _C)
    samp = sampling.reshape(_N * 2)
    mesh = plsc.VectorSubcoreMesh(
        core_axis_name="c", subcore_axis_name="s",
        num_cores=_NC, num_subcores=_NS)
    run = pl.kernel(
        _sc_body,
        out_type=jax.ShapeDtypeStruct((_N, _C), jnp.float32),
        mesh=mesh,
        scratch_types=[
            pltpu.VMEM((_CHUNK * 2,), jnp.float32),
            pltpu.VMEM((_CHUNK,), jnp.int32),
            pltpu.VMEM((_CHUNK,), jnp.int32),
            pltpu.VMEM((_CHUNK,), jnp.int32),
            pltpu.VMEM((_CHUNK,), jnp.int32),
            pltpu.VMEM((_CHUNK,), jnp.float32),
            pltpu.VMEM((_CHUNK,), jnp.float32),
            pltpu.VMEM((_CHUNK, _C), jnp.float32),
            pltpu.VMEM((_CHUNK, _C), jnp.float32),
            pltpu.VMEM((_CHUNK, _C), jnp.float32),
            pltpu.VMEM((_CHUNK, _C), jnp.float32),
            pltpu.VMEM((_CHUNK, _C), jnp.float32),
            pltpu.SemaphoreType.DMA,
        ],
    )
    out = run(table, samp)
    return out.reshape(_B, _H, _W, _C)


# speedup vs baseline: 1.1802x; 1.1802x over previous
import jax
import jax.numpy as jnp
from jax import lax
from jax.experimental import pallas as pl
from jax.experimental.pallas import tpu as pltpu
from jax.experimental.pallas import tpu_sc as plsc

_B, _H, _W, _C = 4, 384, 384, 96
_HW = _H * _W
_N = _B * _HW
_NC, _NS = 2, 16
_NTILES = _NC * _NS
_PPT = _N // _NTILES
_CHUNK = 128
_STEPS = _PPT // _CHUNK
_GROUPS = _C // 16


def _sc_body(table, sxh, syh, out, sx_v, sy_v, idx0, idx1, idx2, idx3,
             fx_v, fy_v, r0, r1, r2, r3, o_v, sem):
    wid = lax.axis_index("s") * _NC + lax.axis_index("c")
    row_base = (wid // 8) * _HW
    p0 = wid * _PPT

    def step(t, carry):
        prow = p0 + t * _CHUNK
        pltpu.sync_copy(sxh.at[pl.ds(prow, _CHUNK)], sx_v)
        pltpu.sync_copy(syh.at[pl.ds(prow, _CHUNK)], sy_v)
        for j in range(_CHUNK // 16):
            sl = pl.ds(16 * j, 16)
            xs = (sx_v[sl] + 1.0) * (0.5 * (_W - 1))
            ys = (sy_v[sl] + 1.0) * (0.5 * (_H - 1))
            xi = jnp.minimum(xs.astype(jnp.int32), _W - 2)
            yi = jnp.minimum(ys.astype(jnp.int32), _H - 2)
            base = row_base + yi * _W + xi
            idx0[sl] = base
            idx1[sl] = base + 1
            idx2[sl] = base + _W
            idx3[sl] = base + (_W + 1)
            fx_v[sl] = xs - xi.astype(jnp.float32)
            fy_v[sl] = ys - yi.astype(jnp.float32)
        cp0 = pltpu.async_copy(table.at[idx0], r0, sem)
        cp1 = pltpu.async_copy(table.at[idx1], r1, sem)
        cp2 = pltpu.async_copy(table.at[idx2], r2, sem)
        cp3 = pltpu.async_copy(table.at[idx3], r3, sem)
        cp0.wait()
        cp1.wait()
        cp2.wait()
        cp3.wait()

        def grp(j, c):
            fxg = fx_v[pl.ds(16 * j, 16)]
            fyg = fy_v[pl.ds(16 * j, 16)]
            for k in range(16):
                i = 16 * j + k
                fx = fxg[k]
                fy = fyg[k]
                for g in range(_GROUPS):
                    gs = pl.ds(g * 16, 16)
                    a0 = r0[i, gs]
                    a1 = r1[i, gs]
                    a2 = r2[i, gs]
                    a3 = r3[i, gs]
                    top = a0 + fx * (a1 - a0)
                    bot = a2 + fx * (a3 - a2)
                    o_v[i, gs] = top + fy * (bot - top)
            return c

        lax.fori_loop(0, _CHUNK // 16, grp, 0)
        pltpu.sync_copy(o_v, out.at[pl.ds(prow, _CHUNK)])
        return carry

    lax.fori_loop(0, _STEPS, step, 0)


@jax.jit
def kernel(images, sampling):
    table = images.reshape(_N, _C)
    sx = sampling[..., 0].reshape(_N)
    sy = sampling[..., 1].reshape(_N)
    mesh = plsc.VectorSubcoreMesh(
        core_axis_name="c", subcore_axis_name="s",
        num_cores=_NC, num_subcores=_NS)
    run = pl.kernel(
        _sc_body,
        out_type=jax.ShapeDtypeStruct((_N, _C), jnp.float32),
        mesh=mesh,
        compiler_params=pltpu.CompilerParams(use_tc_tiling_on_sc=False),
        scratch_types=[
            pltpu.VMEM((_CHUNK,), jnp.float32),
            pltpu.VMEM((_CHUNK,), jnp.float32),
            pltpu.VMEM((_CHUNK,), jnp.int32),
            pltpu.VMEM((_CHUNK,), jnp.int32),
            pltpu.VMEM((_CHUNK,), jnp.int32),
            pltpu.VMEM((_CHUNK,), jnp.int32),
            pltpu.VMEM((_CHUNK,), jnp.float32),
            pltpu.VMEM((_CHUNK,), jnp.float32),
            pltpu.VMEM((_CHUNK, _C), jnp.float32),
            pltpu.VMEM((_CHUNK, _C), jnp.float32),
            pltpu.VMEM((_CHUNK, _C), jnp.float32),
            pltpu.VMEM((_CHUNK, _C), jnp.float32),
            pltpu.VMEM((_CHUNK, _C), jnp.float32),
            pltpu.SemaphoreType.DMA,
        ],
    )
    out = run(table, sx, sy)
    return out.reshape(_B, _H, _W, _C)
